# Initial kernel scaffold; baseline (speedup 1.0000x reference)
import jax
import jax.numpy as jnp
from jax.experimental import pallas as pl


def kernel(src_image, flow_src_to_tar):
    raise NotImplementedError("write your pallas kernel here")


# speedup vs baseline: 102.1758x; 102.1758x over previous
import jax
import jax.numpy as jnp
from jax import lax
from jax.experimental import pallas as pl
from jax.experimental.pallas import tpu as pltpu
from jax.experimental.pallas import tpu_sc as plsc

B, C, H, W = 4, 3, 512, 512
HW = H * W
NC, NS, LANES = 2, 16, 16
PIX_PER_TILE = HW // NS
CH = 4096
CROWS = CH // W
NCHUNK = PIX_PER_TILE // CH
WPAD = 6
WR = CROWS + 2 * WPAD
WG = WR * (W // 128)
PLANE_G = HW // 128
OBLK = 32


def _body(src_hbm, flow_hbm, zeros_hbm, out_hbm, mask_hbm,
          acc, fxb, fyb, rb, gb, bb, winacc,
          idx0, idx1, idx2, idx3, sb_lin, sb_m, sb_val, rowbuf, idxs):
    c = lax.axis_index("c")
    s = lax.axis_index("s")
    tile_base = s * PIX_PER_TILE
    idxd = (idx0, idx1, idx2, idx3)

    def m8(v):
        return pl.multiple_of(v, 8)

    iota_i = lax.iota(jnp.int32, 16)
    iota_f = iota_i.astype(jnp.float32)
    zf16 = jnp.zeros((16,), jnp.float32)

    def vload(buf, i16):
        return buf[i16 >> 7, pl.ds(i16 & 127, 16)]

    @pl.loop(0, 96)
    def _zrb(i):
        rowbuf[4 + (i >> 3), pl.ds((i & 7) * 16, 16)] = zf16

    for bi in range(2):
        b = c * 2 + bi

        pltpu.sync_copy(zeros_hbm, acc.at[pl.ds(m8(s * (4 * PLANE_G // NS)),
                                                4 * PLANE_G // NS)])
        plsc.subcore_barrier()

        @pl.loop(0, NCHUNK)
        def _chunk(k):
            pix0 = tile_base + k * CH
            y0c = pix0 >> 9
            wy0 = y0c - WPAD

            frow = (b * 2 * HW + pix0) >> 7
            pltpu.sync_copy(flow_hbm.at[pl.ds(m8(frow), CH // 128)], fxb)
            pltpu.sync_copy(flow_hbm.at[pl.ds(m8(frow + HW // 128), CH // 128)],
                            fyb)
            srow = (b * C * HW + pix0) >> 7
            pltpu.sync_copy(src_hbm.at[pl.ds(m8(srow), CH // 128)], rb)
            pltpu.sync_copy(src_hbm.at[pl.ds(m8(srow + HW // 128), CH // 128)],
                            gb)
            pltpu.sync_copy(src_hbm.at[pl.ds(m8(srow + 2 * HW // 128), CH // 128)],
                            bb)

            @pl.loop(0, WG * 8)
            def _zw(i):
                g = i >> 3
                col = (i & 7) * 16
                winacc[0, g, pl.ds(col, 16)] = zf16
                winacc[1, g, pl.ds(col, 16)] = zf16
                winacc[2, g, pl.ds(col, 16)] = zf16
                winacc[3, g, pl.ds(col, 16)] = zf16

            for j in range(WG // 16):
                base = j * 16 + iota_i
                rr = wy0 + (base >> 2)
                gq = jnp.clip(rr, 0, H - 1) * 4 + (base & 3)
                for f in range(4):
                    idxd[f][pl.ds(j * 16, 16)] = f * PLANE_G + gq

            @pl.loop(0, CH // 16)
            def _vec(i):
                i16 = i * 16
                fx = vload(fxb, i16)
                fy = vload(fyb, i16)
                p0 = pix0 + i16
                y0 = p0 >> 9
                x0 = p0 & (W - 1)
                tx = x0.astype(jnp.float32) + iota_f + fx
                ty = y0.astype(jnp.float32) + fy

                def ifloor(t):
                    ti = t.astype(jnp.int32)
                    tf = ti.astype(jnp.float32)
                    return jnp.where(tf > t, ti - 1, ti)

                cx = ifloor(tx + 0.5)
                cy = ifloor(ty + 0.5)
                ex = cx.astype(jnp.float32) - tx
                ey = cy.astype(jnp.float32) - ty

                wxs, nxs, wys, nys, wrow = [], [], [], [], []
                for d in (-1, 0, 1):
                    ddx = ex + jnp.float32(d)
                    wx = jnp.exp(-(ddx * ddx))
                    nx = cx + d
                    wx = jnp.where((nx >= 0) & (nx <= W - 1), wx, 0.0)
                    wxs.append(wx)
                    nxs.append(jnp.clip(nx, 0, W - 1))

                    ddy = ey + jnp.float32(d)
                    wy = jnp.exp(-(ddy * ddy))
                    ny = cy + d
                    wy = jnp.where((ny >= 0) & (ny <= H - 1), wy, 0.0)
                    wys.append(wy)
                    nyc = jnp.clip(ny, 0, H - 1)
                    nys.append(ny)
                    wrow.append(jnp.clip(nyc - wy0, 0, WR - 1) * W)

                r = vload(rb, i16)
                g = vload(gb, i16)
                bch = vload(bb, i16)

                ovflag = jnp.zeros((16,), jnp.bool_)
                for oy in range(3):
                    iw = (nys[oy] >= wy0) & (nys[oy] < wy0 + WR)
                    for ox in range(3):
                        w = wys[oy] * wxs[ox]
                        wf = jnp.where(iw, w, 0.0)
                        lidx = wrow[oy] + nxs[ox]
                        grp = lidx >> 7
                        pos = lidx & 127
                        plsc.addupdate_scatter(winacc, [0 * grp, grp, pos],
                                               wf * r)
                        plsc.addupdate_scatter(winacc, [0 * grp + 1, grp, pos],
                                               wf * g)
                        plsc.addupdate_scatter(winacc, [0 * grp + 2, grp, pos],
                                               wf * bch)
                        plsc.addupdate_scatter(winacc, [0 * grp + 3, grp, pos],
                                               wf)
                        ovflag = ovflag | ((~iw) & (w > 0.0))

                @pl.when(jnp.any(ovflag))
                def _slow():
                    for oy in range(3):
                        iw = (nys[oy] >= wy0) & (nys[oy] < wy0 + WR)
                        for ox in range(3):
                            w = wys[oy] * wxs[ox]
                            ov = (~iw) & (w > 0.0)

                            @pl.when(jnp.any(ov))
                            def _ofs():
                                lin = (jnp.clip(nys[oy], 0, H - 1) * W
                                       + nxs[ox])
                                wv = jnp.where(ov, w, 0.0)
                                sb_lin[pl.ds(0, 16)] = lin
                                sb_m[pl.ds(0, 16)] = ov.astype(jnp.int32)
                                sb_val[pl.ds(0, 16)] = wv * r
                                sb_val[pl.ds(16, 16)] = wv * g
                                sb_val[pl.ds(32, 16)] = wv * bch
                                sb_val[pl.ds(48, 16)] = wv

                                @pl.loop(0, 16)
                                def _lane(l):
                                    mv = sb_m[pl.ds(l, 16)][0]

                                    @pl.when(mv > 0)
                                    def _do():
                                        li = sb_lin[pl.ds(l, 16)][0]
                                        grow = li >> 7
                                        posl = li & 127
                                        lane = posl & 15
                                        colb = posl - lane
                                        idxv = jnp.full((16,), 0, jnp.int32)
                                        for f in range(4):
                                            for cc in range(8):
                                                rowbuf[f, pl.ds(cc * 16,
                                                                16)] = zf16
                                            vv = sb_val[pl.ds(f * 16 + l,
                                                              16)][0]
                                            rowbuf[f, pl.ds(colb, 16)] = (
                                                jnp.where(iota_i == lane,
                                                          vv, 0.0))
                                            idxv = jnp.where(
                                                iota_i == f,
                                                f * PLANE_G + grow, idxv)
                                        idxs[pl.ds(0, 16)] = idxv
                                        pltpu.sync_copy(rowbuf,
                                                        acc.at[idxs],
                                                        add=True)

            pltpu.sync_copy(winacc.at[0], acc.at[idx0], add=True)
            pltpu.sync_copy(winacc.at[1], acc.at[idx1], add=True)
            pltpu.sync_copy(winacc.at[2], acc.at[idx2], add=True)
            pltpu.sync_copy(winacc.at[3], acc.at[idx3], add=True)

        plsc.subcore_barrier()

        @pl.loop(0, PIX_PER_TILE // (OBLK * 128))
        def _out(blk):
            g0 = s * (PIX_PER_TILE // 128) + blk * OBLK
            pltpu.sync_copy(acc.at[pl.ds(m8(g0), OBLK)], rb)
            pltpu.sync_copy(acc.at[pl.ds(m8(PLANE_G + g0), OBLK)], gb)
            pltpu.sync_copy(acc.at[pl.ds(m8(2 * PLANE_G + g0), OBLK)], bb)
            pltpu.sync_copy(acc.at[pl.ds(m8(3 * PLANE_G + g0), OBLK)], fxb)

            @pl.loop(0, OBLK * 8)
            def _norm(i):
                gg = i >> 3
                col = (i & 7) * 16
                ws = fxb[gg, pl.ds(col, 16)]
                den = ws + 1e-8
                rb[gg, pl.ds(col, 16)] = rb[gg, pl.ds(col, 16)] / den
                gb[gg, pl.ds(col, 16)] = gb[gg, pl.ds(col, 16)] / den
                bb[gg, pl.ds(col, 16)] = bb[gg, pl.ds(col, 16)] / den
                fyb[gg, pl.ds(col, 16)] = jnp.where(ws > 0.0, 1.0, 0.0)

            obase = b * C * (HW // 128) + g0
            pltpu.sync_copy(rb, out_hbm.at[pl.ds(m8(obase), OBLK)])
            pltpu.sync_copy(gb, out_hbm.at[pl.ds(m8(obase + HW // 128), OBLK)])
            pltpu.sync_copy(bb, out_hbm.at[pl.ds(m8(obase + 2 * HW // 128),
                                                 OBLK)])
            pltpu.sync_copy(fyb, mask_hbm.at[pl.ds(m8(b * (HW // 128) + g0),
                                                   OBLK)])

        plsc.subcore_barrier()


_splat_call = pl.kernel(
    _body,
    out_type=(
        jax.ShapeDtypeStruct((B * C * HW // 128, 128), jnp.float32),
        jax.ShapeDtypeStruct((B * HW // 128, 128), jnp.float32),
    ),
    mesh=plsc.VectorSubcoreMesh(
        core_axis_name="c", subcore_axis_name="s",
        num_cores=NC, num_subcores=NS,
    ),
    compiler_params=pltpu.CompilerParams(needs_layout_passes=False),
    scratch_types=[
        pltpu.VMEM_SHARED((4 * PLANE_G, 128), jnp.float32),
        pltpu.VMEM((CH // 128, 128), jnp.float32),
        pltpu.VMEM((CH // 128, 128), jnp.float32),
        pltpu.VMEM((OBLK, 128), jnp.float32),
        pltpu.VMEM((OBLK, 128), jnp.float32),
        pltpu.VMEM((OBLK, 128), jnp.float32),
        pltpu.VMEM((4, WG, 128), jnp.float32),
        pltpu.VMEM((WG,), jnp.int32),
        pltpu.VMEM((WG,), jnp.int32),
        pltpu.VMEM((WG,), jnp.int32),
        pltpu.VMEM((WG,), jnp.int32),
        pltpu.VMEM((32,), jnp.int32),
        pltpu.VMEM((32,), jnp.int32),
        pltpu.VMEM((80,), jnp.float32),
        pltpu.VMEM((16, 128), jnp.float32),
        pltpu.VMEM((16,), jnp.int32),
    ],
)


def kernel(src_image, flow_src_to_tar):
    src_flat = src_image.reshape(-1, 128)
    flow_flat = flow_src_to_tar.reshape(-1, 128)
    zeros = jnp.zeros((4 * PLANE_G // NS, 128), jnp.float32)
    out_flat, mask_flat = _splat_call(src_flat, flow_flat, zeros)
    out = out_flat.reshape(B, C, H, W)
    mask = mask_flat.reshape(B, 1, H, W)
    return out, mask


# speedup vs baseline: 111.3685x; 1.0900x over previous
import jax
import jax.numpy as jnp
from jax import lax
from jax.experimental import pallas as pl
from jax.experimental.pallas import tpu as pltpu
from jax.experimental.pallas import tpu_sc as plsc

B, C, H, W = 4, 3, 512, 512
HW = H * W
NC, NS, LANES = 2, 16, 16
PIX_PER_TILE = HW // NS
CH = 4096
CROWS = CH // W
NCHUNK = PIX_PER_TILE // CH
WPAD = 6
WR = CROWS + 2 * WPAD
WG = WR * (W // 128)
PLANE_G = HW // 128
OBLK = 32


def _body(src_hbm, flow_hbm, zeros_hbm, out_hbm, mask_hbm,
          acc, fxb, fyb, rb, gb, bb, winacc,
          idx0, idx1, idx2, idx3, sb_lin, sb_m, sb_val, rowbuf, idxs,
          dsem, isem):
    c = lax.axis_index("c")
    s = lax.axis_index("s")
    tile_base = s * PIX_PER_TILE
    idxd = (idx0, idx1, idx2, idx3)

    def m8(v):
        return pl.multiple_of(v, 8)

    iota_i = lax.iota(jnp.int32, 16)
    iota_f = iota_i.astype(jnp.float32)
    zf16 = jnp.zeros((16,), jnp.float32)

    def vload(buf, i16):
        return buf[i16 >> 7, pl.ds(i16 & 127, 16)]

    @pl.loop(0, 96)
    def _zrb(i):
        rowbuf[4 + (i >> 3), pl.ds((i & 7) * 16, 16)] = zf16

    for bi in range(2):
        b = c * 2 + bi

        pltpu.sync_copy(zeros_hbm, acc.at[pl.ds(m8(s * (4 * PLANE_G // NS)),
                                                4 * PLANE_G // NS)])
        plsc.subcore_barrier()

        @pl.loop(0, NCHUNK)
        def _chunk(k):
            pix0 = tile_base + k * CH
            y0c = pix0 >> 9
            wy0 = y0c - WPAD

            frow = (b * 2 * HW + pix0) >> 7
            pltpu.async_copy(flow_hbm.at[pl.ds(m8(frow), CH // 128)], fxb,
                             isem)
            pltpu.async_copy(flow_hbm.at[pl.ds(m8(frow + HW // 128),
                                               CH // 128)], fyb, isem)
            srow = (b * C * HW + pix0) >> 7
            pltpu.async_copy(src_hbm.at[pl.ds(m8(srow), CH // 128)], rb, isem)
            pltpu.async_copy(src_hbm.at[pl.ds(m8(srow + HW // 128),
                                              CH // 128)], gb, isem)
            pltpu.async_copy(src_hbm.at[pl.ds(m8(srow + 2 * HW // 128),
                                              CH // 128)], bb, isem)

            @pl.when(k > 0)
            def _wd():
                for f in range(4):
                    pltpu.make_async_copy(winacc.at[f], acc.at[idxd[f]],
                                          dsem).wait()

            @pl.loop(0, WG * 8)
            def _zw(i):
                g = i >> 3
                col = (i & 7) * 16
                winacc[0, g, pl.ds(col, 16)] = zf16
                winacc[1, g, pl.ds(col, 16)] = zf16
                winacc[2, g, pl.ds(col, 16)] = zf16
                winacc[3, g, pl.ds(col, 16)] = zf16

            for j in range(WG // 16):
                base = j * 16 + iota_i
                rr = wy0 + (base >> 2)
                gq = jnp.clip(rr, 0, H - 1) * 4 + (base & 3)
                for f in range(4):
                    idxd[f][pl.ds(j * 16, 16)] = f * PLANE_G + gq

            pltpu.make_async_copy(flow_hbm.at[pl.ds(0, CH // 128)], fxb,
                                  isem).wait()
            pltpu.make_async_copy(flow_hbm.at[pl.ds(0, CH // 128)], fyb,
                                  isem).wait()
            pltpu.make_async_copy(src_hbm.at[pl.ds(0, CH // 128)], rb,
                                  isem).wait()
            pltpu.make_async_copy(src_hbm.at[pl.ds(0, CH // 128)], gb,
                                  isem).wait()
            pltpu.make_async_copy(src_hbm.at[pl.ds(0, CH // 128)], bb,
                                  isem).wait()

            @pl.loop(0, CH // 16)
            def _vec(i):
                i16 = i * 16
                fx = vload(fxb, i16)
                fy = vload(fyb, i16)
                p0 = pix0 + i16
                y0 = p0 >> 9
                x0 = p0 & (W - 1)
                tx = x0.astype(jnp.float32) + iota_f + fx
                ty = y0.astype(jnp.float32) + fy

                def ifloor(t):
                    ti = t.astype(jnp.int32)
                    tf = ti.astype(jnp.float32)
                    return jnp.where(tf > t, ti - 1, ti)

                cx = ifloor(tx + 0.5)
                cy = ifloor(ty + 0.5)
                ex = cx.astype(jnp.float32) - tx
                ey = cy.astype(jnp.float32) - ty

                wxs, nxs, wys, nys, wrow = [], [], [], [], []
                for d in (-1, 0, 1):
                    ddx = ex + jnp.float32(d)
                    wx = jnp.exp(-(ddx * ddx))
                    nx = cx + d
                    wx = jnp.where((nx >= 0) & (nx <= W - 1), wx, 0.0)
                    wxs.append(wx)
                    nxs.append(jnp.clip(nx, 0, W - 1))

                    ddy = ey + jnp.float32(d)
                    wy = jnp.exp(-(ddy * ddy))
                    ny = cy + d
                    wy = jnp.where((ny >= 0) & (ny <= H - 1), wy, 0.0)
                    wys.append(wy)
                    nyc = jnp.clip(ny, 0, H - 1)
                    nys.append(ny)
                    wrow.append(jnp.clip(nyc - wy0, 0, WR - 1) * W)

                r = vload(rb, i16)
                g = vload(gb, i16)
                bch = vload(bb, i16)

                ovflag = jnp.zeros((16,), jnp.bool_)
                for oy in range(3):
                    iw = (nys[oy] >= wy0) & (nys[oy] < wy0 + WR)
                    for ox in range(3):
                        w = wys[oy] * wxs[ox]
                        wf = jnp.where(iw, w, 0.0)
                        lidx = wrow[oy] + nxs[ox]
                        grp = lidx >> 7
                        pos = lidx & 127
                        plsc.addupdate_scatter(winacc, [0 * grp, grp, pos],
                                               wf * r)
                        plsc.addupdate_scatter(winacc, [0 * grp + 1, grp, pos],
                                               wf * g)
                        plsc.addupdate_scatter(winacc, [0 * grp + 2, grp, pos],
                                               wf * bch)
                        plsc.addupdate_scatter(winacc, [0 * grp + 3, grp, pos],
                                               wf)
                        ovflag = ovflag | ((~iw) & (w > 0.0))

                @pl.when(jnp.any(ovflag))
                def _slow():
                    for oy in range(3):
                        iw = (nys[oy] >= wy0) & (nys[oy] < wy0 + WR)
                        for ox in range(3):
                            w = wys[oy] * wxs[ox]
                            ov = (~iw) & (w > 0.0)

                            @pl.when(jnp.any(ov))
                            def _ofs():
                                lin = (jnp.clip(nys[oy], 0, H - 1) * W
                                       + nxs[ox])
                                wv = jnp.where(ov, w, 0.0)
                                sb_lin[pl.ds(0, 16)] = lin
                                sb_m[pl.ds(0, 16)] = ov.astype(jnp.int32)
                                sb_val[pl.ds(0, 16)] = wv * r
                                sb_val[pl.ds(16, 16)] = wv * g
                                sb_val[pl.ds(32, 16)] = wv * bch
                                sb_val[pl.ds(48, 16)] = wv

                                @pl.loop(0, 16)
                                def _lane(l):
                                    mv = sb_m[pl.ds(l, 16)][0]

                                    @pl.when(mv > 0)
                                    def _do():
                                        li = sb_lin[pl.ds(l, 16)][0]
                                        grow = li >> 7
                                        posl = li & 127
                                        lane = posl & 15
                                        colb = posl - lane
                                        idxv = jnp.full((16,), 0, jnp.int32)
                                        for f in range(4):
                                            for cc in range(8):
                                                rowbuf[f, pl.ds(cc * 16,
                                                                16)] = zf16
                                            vv = sb_val[pl.ds(f * 16 + l,
                                                              16)][0]
                                            rowbuf[f, pl.ds(colb, 16)] = (
                                                jnp.where(iota_i == lane,
                                                          vv, 0.0))
                                            idxv = jnp.where(
                                                iota_i == f,
                                                f * PLANE_G + grow, idxv)
                                        idxs[pl.ds(0, 16)] = idxv
                                        pltpu.sync_copy(rowbuf,
                                                        acc.at[idxs],
                                                        add=True)

            pltpu.async_copy(winacc.at[0], acc.at[idx0], dsem, add=True)
            pltpu.async_copy(winacc.at[1], acc.at[idx1], dsem, add=True)
            pltpu.async_copy(winacc.at[2], acc.at[idx2], dsem, add=True)
            pltpu.async_copy(winacc.at[3], acc.at[idx3], dsem, add=True)

        for f in range(4):
            pltpu.make_async_copy(winacc.at[f], acc.at[idxd[f]], dsem).wait()

        plsc.subcore_barrier()

        @pl.loop(0, PIX_PER_TILE // (OBLK * 128))
        def _out(blk):
            g0 = s * (PIX_PER_TILE // 128) + blk * OBLK
            pltpu.sync_copy(acc.at[pl.ds(m8(g0), OBLK)], rb)
            pltpu.sync_copy(acc.at[pl.ds(m8(PLANE_G + g0), OBLK)], gb)
            pltpu.sync_copy(acc.at[pl.ds(m8(2 * PLANE_G + g0), OBLK)], bb)
            pltpu.sync_copy(acc.at[pl.ds(m8(3 * PLANE_G + g0), OBLK)], fxb)

            @pl.loop(0, OBLK * 8)
            def _norm(i):
                gg = i >> 3
                col = (i & 7) * 16
                ws = fxb[gg, pl.ds(col, 16)]
                den = ws + 1e-8
                rb[gg, pl.ds(col, 16)] = rb[gg, pl.ds(col, 16)] / den
                gb[gg, pl.ds(col, 16)] = gb[gg, pl.ds(col, 16)] / den
                bb[gg, pl.ds(col, 16)] = bb[gg, pl.ds(col, 16)] / den
                fyb[gg, pl.ds(col, 16)] = jnp.where(ws > 0.0, 1.0, 0.0)

            obase = b * C * (HW // 128) + g0
            pltpu.sync_copy(rb, out_hbm.at[pl.ds(m8(obase), OBLK)])
            pltpu.sync_copy(gb, out_hbm.at[pl.ds(m8(obase + HW // 128), OBLK)])
            pltpu.sync_copy(bb, out_hbm.at[pl.ds(m8(obase + 2 * HW // 128),
                                                 OBLK)])
            pltpu.sync_copy(fyb, mask_hbm.at[pl.ds(m8(b * (HW // 128) + g0),
                                                   OBLK)])

        plsc.subcore_barrier()


_splat_call = pl.kernel(
    _body,
    out_type=(
        jax.ShapeDtypeStruct((B * C * HW // 128, 128), jnp.float32),
        jax.ShapeDtypeStruct((B * HW // 128, 128), jnp.float32),
    ),
    mesh=plsc.VectorSubcoreMesh(
        core_axis_name="c", subcore_axis_name="s",
        num_cores=NC, num_subcores=NS,
    ),
    compiler_params=pltpu.CompilerParams(needs_layout_passes=False),
    scratch_types=[
        pltpu.VMEM_SHARED((4 * PLANE_G, 128), jnp.float32),
        pltpu.VMEM((CH // 128, 128), jnp.float32),
        pltpu.VMEM((CH // 128, 128), jnp.float32),
        pltpu.VMEM((OBLK, 128), jnp.float32),
        pltpu.VMEM((OBLK, 128), jnp.float32),
        pltpu.VMEM((OBLK, 128), jnp.float32),
        pltpu.VMEM((4, WG, 128), jnp.float32),
        pltpu.VMEM((WG,), jnp.int32),
        pltpu.VMEM((WG,), jnp.int32),
        pltpu.VMEM((WG,), jnp.int32),
        pltpu.VMEM((WG,), jnp.int32),
        pltpu.VMEM((32,), jnp.int32),
        pltpu.VMEM((32,), jnp.int32),
        pltpu.VMEM((80,), jnp.float32),
        pltpu.VMEM((16, 128), jnp.float32),
        pltpu.VMEM((16,), jnp.int32),
        pltpu.SemaphoreType.DMA,
        pltpu.SemaphoreType.DMA,
    ],
)


def kernel(src_image, flow_src_to_tar):
    src_flat = src_image.reshape(-1, 128)
    flow_flat = flow_src_to_tar.reshape(-1, 128)
    zeros = jnp.zeros((4 * PLANE_G // NS, 128), jnp.float32)
    out_flat, mask_flat = _splat_call(src_flat, flow_flat, zeros)
    out = out_flat.reshape(B, C, H, W)
    mask = mask_flat.reshape(B, 1, H, W)
    return out, mask


# speedup vs baseline: 114.7901x; 1.0307x over previous
import jax
import jax.numpy as jnp
from jax import lax
from jax.experimental import pallas as pl
from jax.experimental.pallas import tpu as pltpu
from jax.experimental.pallas import tpu_sc as plsc

B, C, H, W = 4, 3, 512, 512
HW = H * W
NC, NS, LANES = 2, 16, 16
PIX_PER_TILE = HW // NS
CH = 2048
CROWS = CH // W
NCHUNK = PIX_PER_TILE // CH
WPAD = 8
WR = 20
WG = WR * (W // 128)
DG = CROWS * (W // 128)
PLANE_G = HW // 128
OBLK = 16


def _body(src_hbm, flow_hbm, zeros_hbm, out_hbm, mask_hbm,
          acc, fxb, fyb, rb, gb, bb, winacc,
          idx0, idx1, idx2, idx3, jdx0, jdx1, jdx2, jdx3,
          sb_lin, sb_m, sb_val, rowbuf, idxs,
          dsem, isem):
    c = lax.axis_index("c")
    s = lax.axis_index("s")
    tile_base = s * PIX_PER_TILE
    idxd = (idx0, idx1, idx2, idx3)
    jdxd = (jdx0, jdx1, jdx2, jdx3)

    def m8(v):
        return pl.multiple_of(v, 8)

    def mod20(v):
        q = (v * 3277) >> 16
        return v - q * 20

    iota_i = lax.iota(jnp.int32, 16)
    iota_f = iota_i.astype(jnp.float32)
    zf16 = jnp.zeros((16,), jnp.float32)

    def vload(buf, i16):
        return buf[i16 >> 7, pl.ds(i16 & 127, 16)]

    @pl.loop(0, 96)
    def _zrb(i):
        rowbuf[4 + (i >> 3), pl.ds((i & 7) * 16, 16)] = zf16

    for bi in range(2):
        b = c * 2 + bi

        pltpu.sync_copy(zeros_hbm, acc.at[pl.ds(m8(s * (4 * PLANE_G // NS)),
                                                4 * PLANE_G // NS)])
        plsc.subcore_barrier()

        @pl.loop(0, WG * 8)
        def _zw(i):
            gz = i >> 3
            col = (i & 7) * 16
            winacc[0, gz, pl.ds(col, 16)] = zf16
            winacc[1, gz, pl.ds(col, 16)] = zf16
            winacc[2, gz, pl.ds(col, 16)] = zf16
            winacc[3, gz, pl.ds(col, 16)] = zf16

        wy0L = s * 32 + (NCHUNK - 1) * CROWS - WPAD
        m0 = mod20(wy0L + 40)
        for j in range(WG // 16):
            gi = j * 16 + iota_i
            t = gi >> 2
            q = gi & 3
            rowi = wy0L + mod20(t - m0 + 40)
            gq = jnp.clip(rowi, 0, H - 1) * 4 + q
            for f in range(4):
                idxd[f][pl.ds(j * 16, 16)] = f * PLANE_G + gq

        @pl.loop(0, NCHUNK)
        def _chunk(k):
            pix0 = tile_base + k * CH
            y0c = pix0 >> 9
            wy0 = y0c - WPAD

            frow = (b * 2 * HW + pix0) >> 7
            pltpu.async_copy(flow_hbm.at[pl.ds(m8(frow), CH // 128)], fxb,
                             isem)
            pltpu.async_copy(flow_hbm.at[pl.ds(m8(frow + HW // 128),
                                               CH // 128)], fyb, isem)
            srow = (b * C * HW + pix0) >> 7
            pltpu.async_copy(src_hbm.at[pl.ds(m8(srow), CH // 128)], rb, isem)
            pltpu.async_copy(src_hbm.at[pl.ds(m8(srow + HW // 128),
                                              CH // 128)], gb, isem)
            pltpu.async_copy(src_hbm.at[pl.ds(m8(srow + 2 * HW // 128),
                                              CH // 128)], bb, isem)

            @pl.when(k > 0)
            def _wd():
                psp = mod20(wy0 - CROWS + 40) * 4
                for f in range(4):
                    pltpu.make_async_copy(
                        winacc.at[f, pl.ds(psp, DG)],
                        acc.at[jdxd[f]], dsem).wait()

                @pl.loop(0, DG * 8)
                def _zs(i):
                    gz = psp + (i >> 3)
                    col = (i & 7) * 16
                    winacc[0, gz, pl.ds(col, 16)] = zf16
                    winacc[1, gz, pl.ds(col, 16)] = zf16
                    winacc[2, gz, pl.ds(col, 16)] = zf16
                    winacc[3, gz, pl.ds(col, 16)] = zf16

            pltpu.make_async_copy(flow_hbm.at[pl.ds(0, CH // 128)], fxb,
                                  isem).wait()
            pltpu.make_async_copy(flow_hbm.at[pl.ds(0, CH // 128)], fyb,
                                  isem).wait()
            pltpu.make_async_copy(src_hbm.at[pl.ds(0, CH // 128)], rb,
                                  isem).wait()
            pltpu.make_async_copy(src_hbm.at[pl.ds(0, CH // 128)], gb,
                                  isem).wait()
            pltpu.make_async_copy(src_hbm.at[pl.ds(0, CH // 128)], bb,
                                  isem).wait()

            @pl.loop(0, CH // 16)
            def _vec(i):
                i16 = i * 16
                fx = vload(fxb, i16)
                fy = vload(fyb, i16)
                p0 = pix0 + i16
                y0 = p0 >> 9
                x0 = p0 & (W - 1)
                tx = x0.astype(jnp.float32) + iota_f + fx
                ty = y0.astype(jnp.float32) + fy

                def ifloor(t):
                    ti = t.astype(jnp.int32)
                    tf = ti.astype(jnp.float32)
                    return jnp.where(tf > t, ti - 1, ti)

                cx = ifloor(tx + 0.5)
                cy = ifloor(ty + 0.5)
                ex = cx.astype(jnp.float32) - tx
                ey = cy.astype(jnp.float32) - ty

                wxs, nxs, wys, nys, wrow = [], [], [], [], []
                for d in (-1, 0, 1):
                    ddx = ex + jnp.float32(d)
                    wx = jnp.exp(-(ddx * ddx))
                    nx = cx + d
                    wx = jnp.where((nx >= 0) & (nx <= W - 1), wx, 0.0)
                    wxs.append(wx)
                    nxs.append(jnp.clip(nx, 0, W - 1))

                    ddy = ey + jnp.float32(d)
                    wy = jnp.exp(-(ddy * ddy))
                    ny = cy + d
                    wy = jnp.where((ny >= 0) & (ny <= H - 1), wy, 0.0)
                    wys.append(wy)
                    nyc = jnp.clip(ny, 0, H - 1)
                    nys.append(ny)
                    wrow.append(mod20(nyc) * W)

                r = vload(rb, i16)
                g = vload(gb, i16)
                bch = vload(bb, i16)

                ovflag = jnp.zeros((16,), jnp.bool_)
                for oy in range(3):
                    iw = (nys[oy] >= wy0) & (nys[oy] < wy0 + WR)
                    for ox in range(3):
                        w = wys[oy] * wxs[ox]
                        wf = jnp.where(iw, w, 0.0)
                        lidx = wrow[oy] + nxs[ox]
                        grp = lidx >> 7
                        pos = lidx & 127
                        plsc.addupdate_scatter(winacc, [0 * grp, grp, pos],
                                               wf * r)
                        plsc.addupdate_scatter(winacc, [0 * grp + 1, grp, pos],
                                               wf * g)
                        plsc.addupdate_scatter(winacc, [0 * grp + 2, grp, pos],
                                               wf * bch)
                        plsc.addupdate_scatter(winacc, [0 * grp + 3, grp, pos],
                                               wf)
                        ovflag = ovflag | ((~iw) & (w > 0.0))

                @pl.when(jnp.any(ovflag))
                def _slow():
                    for oy in range(3):
                        iw = (nys[oy] >= wy0) & (nys[oy] < wy0 + WR)
                        for ox in range(3):
                            w = wys[oy] * wxs[ox]
                            ov = (~iw) & (w > 0.0)

                            @pl.when(jnp.any(ov))
                            def _ofs():
                                lin = (jnp.clip(nys[oy], 0, H - 1) * W
                                       + nxs[ox])
                                wv = jnp.where(ov, w, 0.0)
                                sb_lin[pl.ds(0, 16)] = lin
                                sb_m[pl.ds(0, 16)] = ov.astype(jnp.int32)
                                sb_val[pl.ds(0, 16)] = wv * r
                                sb_val[pl.ds(16, 16)] = wv * g
                                sb_val[pl.ds(32, 16)] = wv * bch
                                sb_val[pl.ds(48, 16)] = wv

                                @pl.loop(0, 16)
                                def _lane(l):
                                    mv = sb_m[pl.ds(l, 16)][0]

                                    @pl.when(mv > 0)
                                    def _do():
                                        li = sb_lin[pl.ds(l, 16)][0]
                                        grow = li >> 7
                                        posl = li & 127
                                        lane = posl & 15
                                        colb = posl - lane
                                        idxv = jnp.full((16,), 0, jnp.int32)
                                        for f in range(4):
                                            for cc in range(8):
                                                rowbuf[f, pl.ds(cc * 16,
                                                                16)] = zf16
                                            vv = sb_val[pl.ds(f * 16 + l,
                                                              16)][0]
                                            rowbuf[f, pl.ds(colb, 16)] = (
                                                jnp.where(iota_i == lane,
                                                          vv, 0.0))
                                            idxv = jnp.where(
                                                iota_i == f,
                                                f * PLANE_G + grow, idxv)
                                        idxs[pl.ds(0, 16)] = idxv
                                        pltpu.sync_copy(rowbuf,
                                                        acc.at[idxs],
                                                        add=True)

            @pl.when(k < NCHUNK - 1)
            def _pd():
                gi = iota_i
                rr = wy0 + (gi >> 2)
                gq = jnp.clip(rr, 0, H - 1) * 4 + (gi & 3)
                for f in range(4):
                    jdxd[f][pl.ds(0, 16)] = f * PLANE_G + gq
                ps = mod20(wy0 + 40) * 4
                for f in range(4):
                    pltpu.async_copy(winacc.at[f, pl.ds(ps, DG)],
                                     acc.at[jdxd[f]], dsem, add=True)

        for f in range(4):
            pltpu.sync_copy(winacc.at[f], acc.at[idxd[f]], add=True)

        plsc.subcore_barrier()

        @pl.loop(0, PIX_PER_TILE // (OBLK * 128))
        def _out(blk):
            g0 = s * (PIX_PER_TILE // 128) + blk * OBLK
            pltpu.sync_copy(acc.at[pl.ds(m8(g0), OBLK)], rb)
            pltpu.sync_copy(acc.at[pl.ds(m8(PLANE_G + g0), OBLK)], gb)
            pltpu.sync_copy(acc.at[pl.ds(m8(2 * PLANE_G + g0), OBLK)], bb)
            pltpu.sync_copy(acc.at[pl.ds(m8(3 * PLANE_G + g0), OBLK)], fxb)

            @pl.loop(0, OBLK * 8)
            def _norm(i):
                gg = i >> 3
                col = (i & 7) * 16
                ws = fxb[gg, pl.ds(col, 16)]
                den = ws + 1e-8
                rb[gg, pl.ds(col, 16)] = rb[gg, pl.ds(col, 16)] / den
                gb[gg, pl.ds(col, 16)] = gb[gg, pl.ds(col, 16)] / den
                bb[gg, pl.ds(col, 16)] = bb[gg, pl.ds(col, 16)] / den
                fyb[gg, pl.ds(col, 16)] = jnp.where(ws > 0.0, 1.0, 0.0)

            obase = b * C * (HW // 128) + g0
            pltpu.sync_copy(rb, out_hbm.at[pl.ds(m8(obase), OBLK)])
            pltpu.sync_copy(gb, out_hbm.at[pl.ds(m8(obase + HW // 128), OBLK)])
            pltpu.sync_copy(bb, out_hbm.at[pl.ds(m8(obase + 2 * HW // 128),
                                                 OBLK)])
            pltpu.sync_copy(fyb, mask_hbm.at[pl.ds(m8(b * (HW // 128) + g0),
                                                   OBLK)])

        plsc.subcore_barrier()


_splat_call = pl.kernel(
    _body,
    out_type=(
        jax.ShapeDtypeStruct((B * C * HW // 128, 128), jnp.float32),
        jax.ShapeDtypeStruct((B * HW // 128, 128), jnp.float32),
    ),
    mesh=plsc.VectorSubcoreMesh(
        core_axis_name="c", subcore_axis_name="s",
        num_cores=NC, num_subcores=NS,
    ),
    compiler_params=pltpu.CompilerParams(needs_layout_passes=False),
    scratch_types=[
        pltpu.VMEM_SHARED((4 * PLANE_G, 128), jnp.float32),
        pltpu.VMEM((CH // 128, 128), jnp.float32),
        pltpu.VMEM((CH // 128, 128), jnp.float32),
        pltpu.VMEM((OBLK, 128), jnp.float32),
        pltpu.VMEM((OBLK, 128), jnp.float32),
        pltpu.VMEM((OBLK, 128), jnp.float32),
        pltpu.VMEM((4, WG, 128), jnp.float32),
        pltpu.VMEM((WG,), jnp.int32),
        pltpu.VMEM((WG,), jnp.int32),
        pltpu.VMEM((WG,), jnp.int32),
        pltpu.VMEM((WG,), jnp.int32),
        pltpu.VMEM((16,), jnp.int32),
        pltpu.VMEM((16,), jnp.int32),
        pltpu.VMEM((16,), jnp.int32),
        pltpu.VMEM((16,), jnp.int32),
        pltpu.VMEM((32,), jnp.int32),
        pltpu.VMEM((32,), jnp.int32),
        pltpu.VMEM((80,), jnp.float32),
        pltpu.VMEM((16, 128), jnp.float32),
        pltpu.VMEM((16,), jnp.int32),
        pltpu.SemaphoreType.DMA,
        pltpu.SemaphoreType.DMA,
    ],
)


def kernel(src_image, flow_src_to_tar):
    src_flat = src_image.reshape(-1, 128)
    flow_flat = flow_src_to_tar.reshape(-1, 128)
    zeros = jnp.zeros((4 * PLANE_G // NS, 128), jnp.float32)
    out_flat, mask_flat = _splat_call(src_flat, flow_flat, zeros)
    out = out_flat.reshape(B, C, H, W)
    mask = mask_flat.reshape(B, 1, H, W)
    return out, mask


# speedup vs baseline: 114.8118x; 1.0002x over previous
import jax
import jax.numpy as jnp
from jax import lax
from jax.experimental import pallas as pl
from jax.experimental.pallas import tpu as pltpu
from jax.experimental.pallas import tpu_sc as plsc

B, C, H, W = 4, 3, 512, 512
HW = H * W
NC, NS, LANES = 2, 16, 16
PIX_PER_TILE = HW // NS
CH = 2048
CROWS = CH // W
NCHUNK = PIX_PER_TILE // CH
WPAD = 8
WR = 20
WG = WR * (W // 128)
DG = CROWS * (W // 128)
PLANE_G = HW // 128
OBLK = 16


def _body(src_hbm, flow_hbm, zeros_hbm, out_hbm, mask_hbm,
          acc, fxb, fyb, rb, gb, bb, winacc,
          idx0, idx1, idx2, idx3, jdx0, jdx1, jdx2, jdx3,
          sb_lin, sb_m, sb_val, rowbuf, idxs,
          dsem, isem):
    c = lax.axis_index("c")
    s = lax.axis_index("s")
    tile_base = s * PIX_PER_TILE
    idxd = (idx0, idx1, idx2, idx3)
    jdxd = (jdx0, jdx1, jdx2, jdx3)

    def m8(v):
        return pl.multiple_of(v, 8)

    def mod20(v):
        q = (v * 3277) >> 16
        return v - q * 20

    iota_i = lax.iota(jnp.int32, 16)
    iota_f = iota_i.astype(jnp.float32)
    zf16 = jnp.zeros((16,), jnp.float32)

    def vload(buf, i16):
        return buf[i16 >> 7, pl.ds(i16 & 127, 16)]

    @pl.loop(0, 96)
    def _zrb(i):
        rowbuf[4 + (i >> 3), pl.ds((i & 7) * 16, 16)] = zf16

    for bi in range(2):
        b = c * 2 + bi

        with jax.named_scope("zero_acc"):
            pltpu.sync_copy(zeros_hbm,
                            acc.at[pl.ds(m8(s * (4 * PLANE_G // NS)),
                                         4 * PLANE_G // NS)])
            plsc.subcore_barrier()

        @pl.loop(0, WG * 8)
        def _zw(i):
            gz = i >> 3
            col = (i & 7) * 16
            winacc[0, gz, pl.ds(col, 16)] = zf16
            winacc[1, gz, pl.ds(col, 16)] = zf16
            winacc[2, gz, pl.ds(col, 16)] = zf16
            winacc[3, gz, pl.ds(col, 16)] = zf16

        wy0L = s * 32 + (NCHUNK - 1) * CROWS - WPAD
        m0 = mod20(wy0L + 40)
        for j in range(WG // 16):
            gi = j * 16 + iota_i
            t = gi >> 2
            q = gi & 3
            rowi = wy0L + mod20(t - m0 + 40)
            gq = jnp.clip(rowi, 0, H - 1) * 4 + q
            for f in range(4):
                idxd[f][pl.ds(j * 16, 16)] = f * PLANE_G + gq

        @pl.loop(0, NCHUNK)
        def _chunk(k):
            pix0 = tile_base + k * CH
            y0c = pix0 >> 9
            wy0 = y0c - WPAD

            frow = (b * 2 * HW + pix0) >> 7
            pltpu.async_copy(flow_hbm.at[pl.ds(m8(frow), CH // 128)], fxb,
                             isem)
            pltpu.async_copy(flow_hbm.at[pl.ds(m8(frow + HW // 128),
                                               CH // 128)], fyb, isem)
            srow = (b * C * HW + pix0) >> 7
            pltpu.async_copy(src_hbm.at[pl.ds(m8(srow), CH // 128)], rb, isem)
            pltpu.async_copy(src_hbm.at[pl.ds(m8(srow + HW // 128),
                                              CH // 128)], gb, isem)
            pltpu.async_copy(src_hbm.at[pl.ds(m8(srow + 2 * HW // 128),
                                              CH // 128)], bb, isem)

            @pl.when(k > 0)
            def _wd():
                psp = mod20(wy0 - CROWS + 40) * 4
                for f in range(4):
                    pltpu.make_async_copy(
                        winacc.at[f, pl.ds(psp, DG)],
                        acc.at[jdxd[f]], dsem).wait()

                @pl.loop(0, DG * 8)
                def _zs(i):
                    gz = psp + (i >> 3)
                    col = (i & 7) * 16
                    winacc[0, gz, pl.ds(col, 16)] = zf16
                    winacc[1, gz, pl.ds(col, 16)] = zf16
                    winacc[2, gz, pl.ds(col, 16)] = zf16
                    winacc[3, gz, pl.ds(col, 16)] = zf16

            pltpu.make_async_copy(flow_hbm.at[pl.ds(0, CH // 128)], fxb,
                                  isem).wait()
            pltpu.make_async_copy(flow_hbm.at[pl.ds(0, CH // 128)], fyb,
                                  isem).wait()
            pltpu.make_async_copy(src_hbm.at[pl.ds(0, CH // 128)], rb,
                                  isem).wait()
            pltpu.make_async_copy(src_hbm.at[pl.ds(0, CH // 128)], gb,
                                  isem).wait()
            pltpu.make_async_copy(src_hbm.at[pl.ds(0, CH // 128)], bb,
                                  isem).wait()

            @pl.loop(0, CH // 16)
            def _vec(i):
                i16 = i * 16
                fx = vload(fxb, i16)
                fy = vload(fyb, i16)
                p0 = pix0 + i16
                y0 = p0 >> 9
                x0 = p0 & (W - 1)
                tx = x0.astype(jnp.float32) + iota_f + fx
                ty = y0.astype(jnp.float32) + fy

                def ifloor(t):
                    ti = t.astype(jnp.int32)
                    tf = ti.astype(jnp.float32)
                    return jnp.where(tf > t, ti - 1, ti)

                cx = ifloor(tx + 0.5)
                cy = ifloor(ty + 0.5)
                ex = cx.astype(jnp.float32) - tx
                ey = cy.astype(jnp.float32) - ty

                wxs, nxs, wys, nys, wrow = [], [], [], [], []
                for d in (-1, 0, 1):
                    ddx = ex + jnp.float32(d)
                    wx = jnp.exp(-(ddx * ddx))
                    nx = cx + d
                    wx = jnp.where((nx >= 0) & (nx <= W - 1), wx, 0.0)
                    wxs.append(wx)
                    nxs.append(jnp.clip(nx, 0, W - 1))

                    ddy = ey + jnp.float32(d)
                    wy = jnp.exp(-(ddy * ddy))
                    ny = cy + d
                    wy = jnp.where((ny >= 0) & (ny <= H - 1), wy, 0.0)
                    wys.append(wy)
                    nyc = jnp.clip(ny, 0, H - 1)
                    nys.append(ny)
                    wrow.append(mod20(nyc) * W)

                r = vload(rb, i16)
                g = vload(gb, i16)
                bch = vload(bb, i16)

                ovflag = jnp.zeros((16,), jnp.bool_)
                for oy in range(3):
                    iw = (nys[oy] >= wy0) & (nys[oy] < wy0 + WR)
                    for ox in range(3):
                        w = wys[oy] * wxs[ox]
                        wf = jnp.where(iw, w, 0.0)
                        lidx = wrow[oy] + nxs[ox]
                        grp = lidx >> 7
                        pos = lidx & 127
                        plsc.addupdate_scatter(winacc, [0 * grp, grp, pos],
                                               wf * r)
                        plsc.addupdate_scatter(winacc, [0 * grp + 1, grp, pos],
                                               wf * g)
                        plsc.addupdate_scatter(winacc, [0 * grp + 2, grp, pos],
                                               wf * bch)
                        plsc.addupdate_scatter(winacc, [0 * grp + 3, grp, pos],
                                               wf)
                        ovflag = ovflag | ((~iw) & (w > 0.0))

                @pl.when(jnp.any(ovflag))
                def _slow():
                    for oy in range(3):
                        iw = (nys[oy] >= wy0) & (nys[oy] < wy0 + WR)
                        for ox in range(3):
                            w = wys[oy] * wxs[ox]
                            ov = (~iw) & (w > 0.0)

                            @pl.when(jnp.any(ov))
                            def _ofs():
                                lin = (jnp.clip(nys[oy], 0, H - 1) * W
                                       + nxs[ox])
                                wv = jnp.where(ov, w, 0.0)
                                sb_lin[pl.ds(0, 16)] = lin
                                sb_m[pl.ds(0, 16)] = ov.astype(jnp.int32)
                                sb_val[pl.ds(0, 16)] = wv * r
                                sb_val[pl.ds(16, 16)] = wv * g
                                sb_val[pl.ds(32, 16)] = wv * bch
                                sb_val[pl.ds(48, 16)] = wv

                                @pl.loop(0, 16)
                                def _lane(l):
                                    mv = sb_m[pl.ds(l, 16)][0]

                                    @pl.when(mv > 0)
                                    def _do():
                                        li = sb_lin[pl.ds(l, 16)][0]
                                        grow = li >> 7
                                        posl = li & 127
                                        lane = posl & 15
                                        colb = posl - lane
                                        idxv = jnp.full((16,), 0, jnp.int32)
                                        for f in range(4):
                                            for cc in range(8):
                                                rowbuf[f, pl.ds(cc * 16,
                                                                16)] = zf16
                                            vv = sb_val[pl.ds(f * 16 + l,
                                                              16)][0]
                                            rowbuf[f, pl.ds(colb, 16)] = (
                                                jnp.where(iota_i == lane,
                                                          vv, 0.0))
                                            idxv = jnp.where(
                                                iota_i == f,
                                                f * PLANE_G + grow, idxv)
                                        idxs[pl.ds(0, 16)] = idxv
                                        pltpu.sync_copy(rowbuf,
                                                        acc.at[idxs],
                                                        add=True)

            @pl.when(k < NCHUNK - 1)
            def _pd():
                gi = iota_i
                rr = wy0 + (gi >> 2)
                gq = jnp.clip(rr, 0, H - 1) * 4 + (gi & 3)
                for f in range(4):
                    jdxd[f][pl.ds(0, 16)] = f * PLANE_G + gq
                ps = mod20(wy0 + 40) * 4
                for f in range(4):
                    pltpu.async_copy(winacc.at[f, pl.ds(ps, DG)],
                                     acc.at[jdxd[f]], dsem, add=True)

        with jax.named_scope("final_drain"):
            for f in range(4):
                pltpu.sync_copy(winacc.at[f], acc.at[idxd[f]], add=True)
            plsc.subcore_barrier()

        @pl.loop(0, PIX_PER_TILE // (OBLK * 128))
        def _out(blk):
            g0 = s * (PIX_PER_TILE // 128) + blk * OBLK
            pltpu.sync_copy(acc.at[pl.ds(m8(g0), OBLK)], rb)
            pltpu.sync_copy(acc.at[pl.ds(m8(PLANE_G + g0), OBLK)], gb)
            pltpu.sync_copy(acc.at[pl.ds(m8(2 * PLANE_G + g0), OBLK)], bb)
            pltpu.sync_copy(acc.at[pl.ds(m8(3 * PLANE_G + g0), OBLK)], fxb)

            @pl.loop(0, OBLK * 8)
            def _norm(i):
                gg = i >> 3
                col = (i & 7) * 16
                ws = fxb[gg, pl.ds(col, 16)]
                den = ws + 1e-8
                rb[gg, pl.ds(col, 16)] = rb[gg, pl.ds(col, 16)] / den
                gb[gg, pl.ds(col, 16)] = gb[gg, pl.ds(col, 16)] / den
                bb[gg, pl.ds(col, 16)] = bb[gg, pl.ds(col, 16)] / den
                fyb[gg, pl.ds(col, 16)] = jnp.where(ws > 0.0, 1.0, 0.0)

            obase = b * C * (HW // 128) + g0
            pltpu.sync_copy(rb, out_hbm.at[pl.ds(m8(obase), OBLK)])
            pltpu.sync_copy(gb, out_hbm.at[pl.ds(m8(obase + HW // 128), OBLK)])
            pltpu.sync_copy(bb, out_hbm.at[pl.ds(m8(obase + 2 * HW // 128),
                                                 OBLK)])
            pltpu.sync_copy(fyb, mask_hbm.at[pl.ds(m8(b * (HW // 128) + g0),
                                                   OBLK)])

        plsc.subcore_barrier()


_splat_call = pl.kernel(
    _body,
    out_type=(
        jax.ShapeDtypeStruct((B * C * HW // 128, 128), jnp.float32),
        jax.ShapeDtypeStruct((B * HW // 128, 128), jnp.float32),
    ),
    mesh=plsc.VectorSubcoreMesh(
        core_axis_name="c", subcore_axis_name="s",
        num_cores=NC, num_subcores=NS,
    ),
    compiler_params=pltpu.CompilerParams(needs_layout_passes=False),
    scratch_types=[
        pltpu.VMEM_SHARED((4 * PLANE_G, 128), jnp.float32),
        pltpu.VMEM((CH // 128, 128), jnp.float32),
        pltpu.VMEM((CH // 128, 128), jnp.float32),
        pltpu.VMEM((OBLK, 128), jnp.float32),
        pltpu.VMEM((OBLK, 128), jnp.float32),
        pltpu.VMEM((OBLK, 128), jnp.float32),
        pltpu.VMEM((4, WG, 128), jnp.float32),
        pltpu.VMEM((WG,), jnp.int32),
        pltpu.VMEM((WG,), jnp.int32),
        pltpu.VMEM((WG,), jnp.int32),
        pltpu.VMEM((WG,), jnp.int32),
        pltpu.VMEM((16,), jnp.int32),
        pltpu.VMEM((16,), jnp.int32),
        pltpu.VMEM((16,), jnp.int32),
        pltpu.VMEM((16,), jnp.int32),
        pltpu.VMEM((32,), jnp.int32),
        pltpu.VMEM((32,), jnp.int32),
        pltpu.VMEM((80,), jnp.float32),
        pltpu.VMEM((16, 128), jnp.float32),
        pltpu.VMEM((16,), jnp.int32),
        pltpu.SemaphoreType.DMA,
        pltpu.SemaphoreType.DMA,
    ],
)


def kernel(src_image, flow_src_to_tar):
    src_flat = src_image.reshape(-1, 128)
    flow_flat = flow_src_to_tar.reshape(-1, 128)
    zeros = jnp.zeros((4 * PLANE_G // NS, 128), jnp.float32)
    out_flat, mask_flat = _splat_call(src_flat, flow_flat, zeros)
    out = out_flat.reshape(B, C, H, W)
    mask = mask_flat.reshape(B, 1, H, W)
    return out, mask
